# Initial kernel scaffold; baseline (speedup 1.0000x reference)
import jax
import jax.numpy as jnp
from jax.experimental import pallas as pl


def kernel(feats, edge_index, kernel_id, W0, Ws, gammas, betas, Wlin, blin):
    raise NotImplementedError("write your pallas kernel here")


# speedup vs baseline: 15.8584x; 15.8584x over previous
import functools

import jax
import jax.numpy as jnp
from jax import lax
from jax.experimental import pallas as pl
from jax.experimental.pallas import tpu as pltpu
from jax.experimental.pallas import tpu_sc as plsc

N = 100000
E = 1600000
K = 27
M = 16
NUM_CLASSES = 20
KM = K * M

NTILES = 32
SCH = 128
GCH = 512
GPB = 7
EBLK = GPB * GCH
NB = 14
ECHUNK = NB * EBLK
NCH = ECHUNK // SCH
EPAD = NTILES * ECHUNK
N_ACC = 100096
DUMMY_ROW = N + 8
ZROWS = N_ACC // 16
OROWS = 6256
OROWS_LAST = N - 15 * OROWS


def _sc_gather_scatter_body(z_hbm, gidx_hbm, dst_hbm, zeros_hbm, out_hbm,
                            gbuf, dbuf, rows, accum, sem):
    cid = lax.axis_index("c")
    sid = lax.axis_index("s")
    wid = cid * 16 + sid
    pltpu.sync_copy(zeros_hbm.at[pl.ds(sid * ZROWS, ZROWS)],
                    accum.at[pl.ds(sid * ZROWS, ZROWS)])
    plsc.subcore_barrier()

    def blk(b, carry):
        pltpu.sync_copy(gidx_hbm.at[wid, b], gbuf)
        pltpu.sync_copy(dst_hbm.at[wid, b], dbuf)

        def body(c, carry2):
            pltpu.async_copy(z_hbm.at[gbuf.at[pl.ds(c * GCH, GCH)]], rows,
                             sem).wait()
            for j in range(GCH // SCH):
                pltpu.sync_copy(rows.at[pl.ds(j * SCH, SCH)],
                                accum.at[dbuf.at[c * (GCH // SCH) + j]],
                                add=True)
            return carry2

        lax.fori_loop(0, GPB, body, 0)
        return carry

    lax.fori_loop(0, NB, blk, 0)
    plsc.subcore_barrier()

    @pl.when(sid < 15)
    def _():
        pltpu.sync_copy(accum.at[pl.ds(sid * OROWS, OROWS)],
                        out_hbm.at[cid, pl.ds(sid * OROWS, OROWS)])

    @pl.when(sid == 15)
    def _():
        pltpu.sync_copy(accum.at[pl.ds(15 * OROWS, OROWS_LAST)],
                        out_hbm.at[cid, pl.ds(15 * OROWS, OROWS_LAST)])


_sc_gather_scatter = functools.partial(
    pl.kernel,
    out_type=jax.ShapeDtypeStruct((2, N, M), jnp.float32),
    mesh=plsc.VectorSubcoreMesh(core_axis_name="c", subcore_axis_name="s"),
    scratch_types=[
        pltpu.VMEM((EBLK,), jnp.int32),
        pltpu.VMEM((EBLK // SCH, SCH), jnp.int32),
        pltpu.VMEM((GCH, M), jnp.float32),
        pltpu.VMEM_SHARED((N_ACC, M), jnp.float32),
        pltpu.SemaphoreType.DMA,
    ],
    compiler_params=pltpu.CompilerParams(use_tc_tiling_on_sc=False),
)(_sc_gather_scatter_body)


def _mm_body(x_ref, w_ref, o_ref):
    o_ref[...] = jnp.dot(x_ref[...], w_ref[...],
                         preferred_element_type=jnp.float32)


def _stats_body(p_ref, o_ref):
    i = pl.program_id(0)

    @pl.when(i == 0)
    def _():
        o_ref[...] = jnp.zeros_like(o_ref)

    s = p_ref[0] + p_ref[1]
    o_ref[...] += jnp.concatenate(
        [jnp.sum(s, 0, keepdims=True), jnp.sum(s * s, 0, keepdims=True)], 0)


def _apply_body(p_ref, st_ref, g_ref, b_ref, w_ref, bias_ref, o_ref):
    s = p_ref[0] + p_ref[1]
    mu = st_ref[0:1, :] * (1.0 / N)
    ex2 = st_ref[1:2, :] * (1.0 / N)
    var = ex2 - mu * mu
    inv = lax.rsqrt(var + 1e-4)
    y = (s - mu) * inv * g_ref[...] + b_ref[...]
    y = jnp.maximum(y, 0.0)
    o_ref[...] = jnp.dot(y, w_ref[...],
                         preferred_element_type=jnp.float32) + bias_ref[...]


TN = 2000
TS = 4000


def _make_mm():
    return pl.pallas_call(
        _mm_body,
        grid=(N // TN,),
        in_specs=[pl.BlockSpec((TN, 8), lambda i: (i, 0)),
                  pl.BlockSpec((8, KM), lambda i: (0, 0))],
        out_specs=pl.BlockSpec((TN, KM), lambda i: (i, 0)),
        out_shape=jax.ShapeDtypeStruct((N, KM), jnp.float32),
    )


def _make_stats():
    return pl.pallas_call(
        _stats_body,
        grid=(N // TS,),
        in_specs=[pl.BlockSpec((2, TS, M), lambda i: (0, i, 0))],
        out_specs=pl.BlockSpec((2, M), lambda i: (0, 0)),
        out_shape=jax.ShapeDtypeStruct((2, M), jnp.float32),
    )


def _make_apply(dout):
    return pl.pallas_call(
        _apply_body,
        grid=(N // TN,),
        in_specs=[pl.BlockSpec((2, TN, M), lambda i: (0, i, 0)),
                  pl.BlockSpec((2, M), lambda i: (0, 0)),
                  pl.BlockSpec((1, M), lambda i: (0, 0)),
                  pl.BlockSpec((1, M), lambda i: (0, 0)),
                  pl.BlockSpec((M, dout), lambda i: (0, 0)),
                  pl.BlockSpec((1, dout), lambda i: (0, 0))],
        out_specs=pl.BlockSpec((TN, dout), lambda i: (i, 0)),
        out_shape=jax.ShapeDtypeStruct((N, dout), jnp.float32),
    )


_mm0 = _make_mm()
_stats = _make_stats()
_apply_mid = _make_apply(KM)
_apply_fin = _make_apply(NUM_CLASSES)


def kernel(feats, edge_index, kernel_id, W0, Ws, gammas, betas, Wlin, blin):
    src = edge_index[0]
    dst = edge_index[1]
    gidx = src * K + kernel_id
    pad = EPAD - E
    gidx_p = jnp.concatenate([gidx, jnp.zeros((pad,), jnp.int32)])
    gidx_p = gidx_p.reshape(NTILES, NB, EBLK)
    dst_p = jnp.concatenate([dst, jnp.full((pad,), DUMMY_ROW, jnp.int32)])
    dst2 = dst_p.reshape(NTILES, NB, EBLK // SCH, SCH)
    zeros_acc = jnp.zeros((N_ACC, M), jnp.float32)

    feats8 = jnp.pad(feats, ((0, 0), (0, 5)))
    w0f = jnp.pad(jnp.transpose(W0, (1, 0, 2)).reshape(3, KM),
                  ((0, 5), (0, 0)))
    wfs = [jnp.transpose(Ws[i], (1, 0, 2)).reshape(M, KM) for i in range(3)]

    z = _mm0(feats8, w0f)
    out = None
    for i in range(4):
        parts = _sc_gather_scatter(z.reshape(N * K, M), gidx_p, dst2,
                                   zeros_acc)
        st = _stats(parts)
        g = gammas[i].reshape(1, M)
        b = betas[i].reshape(1, M)
        if i < 3:
            z = _apply_mid(parts, st, g, b, wfs[i],
                           jnp.zeros((1, KM), jnp.float32))
        else:
            out = _apply_fin(parts, st, g, b, Wlin,
                             blin.reshape(1, NUM_CLASSES))
    return out


# speedup vs baseline: 18.1757x; 1.1461x over previous
import functools

import jax
import jax.numpy as jnp
from jax import lax
from jax.experimental import pallas as pl
from jax.experimental.pallas import tpu as pltpu
from jax.experimental.pallas import tpu_sc as plsc

N = 100000
E = 1600000
K = 27
M = 16
NUM_CLASSES = 20
KM = K * M

NTILES = 32
SCH = 128
GCH = 512
GPB = 7
EBLK = GPB * GCH
NB = 14
ECHUNK = NB * EBLK
NCH = ECHUNK // SCH
EPAD = NTILES * ECHUNK
N_ACC = 100096
DUMMY_ROW = N + 8
ZROWS = N_ACC // 16
OROWS = 6256
OROWS_LAST = N - 15 * OROWS


def _sc_gather_scatter_body(z_hbm, gidx_hbm, dst_hbm, zeros_hbm, out_hbm,
                            gbuf, dbuf, rows, accum, sr0, sr1, si0, si1):
    cid = lax.axis_index("c")
    sid = lax.axis_index("s")
    wid = cid * 16 + sid
    sr = (sr0, sr1)
    si = (si0, si1)
    pltpu.sync_copy(zeros_hbm.at[pl.ds(sid * ZROWS, ZROWS)],
                    accum.at[pl.ds(sid * ZROWS, ZROWS)])
    plsc.subcore_barrier()

    def fire_gather(islot, c, rslot):
        pltpu.async_copy(z_hbm.at[gbuf.at[islot, pl.ds(c * GCH, GCH)]],
                         rows.at[rslot], sr[rslot])

    def wait_gather(islot, c, rslot):
        pltpu.make_async_copy(z_hbm.at[gbuf.at[islot, pl.ds(c * GCH, GCH)]],
                              rows.at[rslot], sr[rslot]).wait()

    pltpu.sync_copy(gidx_hbm.at[wid, 0], gbuf.at[0])
    pltpu.sync_copy(dst_hbm.at[wid, 0], dbuf.at[0])
    fire_gather(0, 0, 0)

    def do_block(bb, s):
        sn = 1 - s

        @pl.when(bb + 1 < NB)
        def _():
            pltpu.async_copy(gidx_hbm.at[wid, bb + 1], gbuf.at[sn], si[sn])
            pltpu.async_copy(dst_hbm.at[wid, bb + 1], dbuf.at[sn], si[sn])

        for c in range(GPB):
            rs = (s + c) % 2
            rsn = 1 - rs
            if c < GPB - 1:
                fire_gather(s, c + 1, rsn)
            else:
                @pl.when(bb + 1 < NB)
                def _():
                    pltpu.make_async_copy(gidx_hbm.at[wid, bb + 1],
                                          gbuf.at[sn], si[sn]).wait()
                    pltpu.make_async_copy(dst_hbm.at[wid, bb + 1],
                                          dbuf.at[sn], si[sn]).wait()
                    fire_gather(sn, 0, rsn)
            wait_gather(s, c, rs)
            for j in range(GCH // SCH):
                pltpu.sync_copy(rows.at[rs, pl.ds(j * SCH, SCH)],
                                accum.at[dbuf.at[s, c * (GCH // SCH) + j]],
                                add=True)

    def pair_body(i, carry):
        do_block(2 * i, 0)
        do_block(2 * i + 1, 1)
        return carry

    lax.fori_loop(0, NB // 2, pair_body, 0)
    plsc.subcore_barrier()

    @pl.when(sid < 15)
    def _():
        pltpu.sync_copy(accum.at[pl.ds(sid * OROWS, OROWS)],
                        out_hbm.at[cid, pl.ds(sid * OROWS, OROWS)])

    @pl.when(sid == 15)
    def _():
        pltpu.sync_copy(accum.at[pl.ds(15 * OROWS, OROWS_LAST)],
                        out_hbm.at[cid, pl.ds(15 * OROWS, OROWS_LAST)])


_sc_gather_scatter = functools.partial(
    pl.kernel,
    out_type=jax.ShapeDtypeStruct((2, N, M), jnp.float32),
    mesh=plsc.VectorSubcoreMesh(core_axis_name="c", subcore_axis_name="s"),
    scratch_types=[
        pltpu.VMEM((2, EBLK), jnp.int32),
        pltpu.VMEM((2, EBLK // SCH, SCH), jnp.int32),
        pltpu.VMEM((2, GCH, M), jnp.float32),
        pltpu.VMEM_SHARED((N_ACC, M), jnp.float32),
        pltpu.SemaphoreType.DMA,
        pltpu.SemaphoreType.DMA,
        pltpu.SemaphoreType.DMA,
        pltpu.SemaphoreType.DMA,
    ],
    compiler_params=pltpu.CompilerParams(use_tc_tiling_on_sc=False),
)(_sc_gather_scatter_body)


def _mm_body(x_ref, w_ref, o_ref):
    o_ref[...] = jnp.dot(x_ref[...], w_ref[...],
                         preferred_element_type=jnp.float32)


def _stats_body(p_ref, o_ref):
    i = pl.program_id(0)

    @pl.when(i == 0)
    def _():
        o_ref[...] = jnp.zeros_like(o_ref)

    s = p_ref[0] + p_ref[1]
    o_ref[...] += jnp.concatenate(
        [jnp.sum(s, 0, keepdims=True), jnp.sum(s * s, 0, keepdims=True)], 0)


def _apply_body(p_ref, st_ref, g_ref, b_ref, w_ref, bias_ref, o_ref):
    s = p_ref[0] + p_ref[1]
    mu = st_ref[0:1, :] * (1.0 / N)
    ex2 = st_ref[1:2, :] * (1.0 / N)
    var = ex2 - mu * mu
    inv = lax.rsqrt(var + 1e-4)
    y = (s - mu) * inv * g_ref[...] + b_ref[...]
    y = jnp.maximum(y, 0.0)
    o_ref[...] = jnp.dot(y, w_ref[...],
                         preferred_element_type=jnp.float32) + bias_ref[...]


TN = 2000
TS = 4000


def _make_mm():
    return pl.pallas_call(
        _mm_body,
        grid=(N // TN,),
        in_specs=[pl.BlockSpec((TN, 8), lambda i: (i, 0)),
                  pl.BlockSpec((8, KM), lambda i: (0, 0))],
        out_specs=pl.BlockSpec((TN, KM), lambda i: (i, 0)),
        out_shape=jax.ShapeDtypeStruct((N, KM), jnp.float32),
    )


def _make_stats():
    return pl.pallas_call(
        _stats_body,
        grid=(N // TS,),
        in_specs=[pl.BlockSpec((2, TS, M), lambda i: (0, i, 0))],
        out_specs=pl.BlockSpec((2, M), lambda i: (0, 0)),
        out_shape=jax.ShapeDtypeStruct((2, M), jnp.float32),
    )


def _make_apply(dout):
    return pl.pallas_call(
        _apply_body,
        grid=(N // TN,),
        in_specs=[pl.BlockSpec((2, TN, M), lambda i: (0, i, 0)),
                  pl.BlockSpec((2, M), lambda i: (0, 0)),
                  pl.BlockSpec((1, M), lambda i: (0, 0)),
                  pl.BlockSpec((1, M), lambda i: (0, 0)),
                  pl.BlockSpec((M, dout), lambda i: (0, 0)),
                  pl.BlockSpec((1, dout), lambda i: (0, 0))],
        out_specs=pl.BlockSpec((TN, dout), lambda i: (i, 0)),
        out_shape=jax.ShapeDtypeStruct((N, dout), jnp.float32),
    )


_mm0 = _make_mm()
_stats = _make_stats()
_apply_mid = _make_apply(KM)
_apply_fin = _make_apply(NUM_CLASSES)


def kernel(feats, edge_index, kernel_id, W0, Ws, gammas, betas, Wlin, blin):
    src = edge_index[0]
    dst = edge_index[1]
    gidx = src * K + kernel_id
    pad = EPAD - E
    gidx_p = jnp.concatenate([gidx, jnp.zeros((pad,), jnp.int32)])
    gidx_p = gidx_p.reshape(NTILES, NB, EBLK)
    dst_p = jnp.concatenate([dst, jnp.full((pad,), DUMMY_ROW, jnp.int32)])
    dst2 = dst_p.reshape(NTILES, NB, EBLK // SCH, SCH)
    zeros_acc = jnp.zeros((N_ACC, M), jnp.float32)

    feats8 = jnp.pad(feats, ((0, 0), (0, 5)))
    w0f = jnp.pad(jnp.transpose(W0, (1, 0, 2)).reshape(3, KM),
                  ((0, 5), (0, 0)))
    wfs = [jnp.transpose(Ws[i], (1, 0, 2)).reshape(M, KM) for i in range(3)]

    z = _mm0(feats8, w0f)
    out = None
    for i in range(4):
        parts = _sc_gather_scatter(z.reshape(N * K, M), gidx_p, dst2,
                                   zeros_acc)
        st = _stats(parts)
        g = gammas[i].reshape(1, M)
        b = betas[i].reshape(1, M)
        if i < 3:
            z = _apply_mid(parts, st, g, b, wfs[i],
                           jnp.zeros((1, KM), jnp.float32))
        else:
            out = _apply_fin(parts, st, g, b, Wlin,
                             blin.reshape(1, NUM_CLASSES))
    return out


# speedup vs baseline: 39.7748x; 2.1884x over previous
import functools

import jax
import jax.numpy as jnp
from jax import lax
from jax.experimental import pallas as pl
from jax.experimental.pallas import tpu as pltpu
from jax.experimental.pallas import tpu_sc as plsc

N = 100000
E = 1600000
K = 27
M = 16
NUM_CLASSES = 20
KM = K * M

NP = 100096
NR8 = NP // 8
NROW_REAL = N * M // 128

NTILES = 32
SCH = 128
GCH = 512
GPB = 7
EBLK = GPB * GCH
NB = 14
ECHUNK = NB * EBLK
EPAD = NTILES * ECHUNK
DUMMY_ROW = N + 8
ZROWS = NP // 16


def _sc_gather_scatter_body(z_hbm, gidx_hbm, dst_hbm, zeros_hbm, out_hbm,
                            gbuf, dbuf, rows, accum, sr0, sr1, si0, si1):
    cid = lax.axis_index("c")
    sid = lax.axis_index("s")
    wid = cid * 16 + sid
    sr = (sr0, sr1)
    si = (si0, si1)
    pltpu.sync_copy(zeros_hbm.at[pl.ds(sid * ZROWS, ZROWS)],
                    accum.at[pl.ds(sid * ZROWS, ZROWS)])
    plsc.subcore_barrier()

    def fire_gather(islot, c, rslot):
        pltpu.async_copy(z_hbm.at[gbuf.at[islot, pl.ds(c * GCH, GCH)]],
                         rows.at[rslot], sr[rslot])

    def wait_gather(islot, c, rslot):
        pltpu.make_async_copy(z_hbm.at[gbuf.at[islot, pl.ds(c * GCH, GCH)]],
                              rows.at[rslot], sr[rslot]).wait()

    pltpu.sync_copy(gidx_hbm.at[wid, 0], gbuf.at[0])
    pltpu.sync_copy(dst_hbm.at[wid, 0], dbuf.at[0])
    fire_gather(0, 0, 0)

    def do_block(bb, s):
        sn = 1 - s

        @pl.when(bb + 1 < NB)
        def _():
            pltpu.async_copy(gidx_hbm.at[wid, bb + 1], gbuf.at[sn], si[sn])
            pltpu.async_copy(dst_hbm.at[wid, bb + 1], dbuf.at[sn], si[sn])

        for c in range(GPB):
            rs = (s + c) % 2
            rsn = 1 - rs
            if c < GPB - 1:
                fire_gather(s, c + 1, rsn)
            else:
                @pl.when(bb + 1 < NB)
                def _():
                    pltpu.make_async_copy(gidx_hbm.at[wid, bb + 1],
                                          gbuf.at[sn], si[sn]).wait()
                    pltpu.make_async_copy(dst_hbm.at[wid, bb + 1],
                                          dbuf.at[sn], si[sn]).wait()
                    fire_gather(sn, 0, rsn)
            wait_gather(s, c, rs)
            for j in range(GCH // SCH):
                pltpu.sync_copy(rows.at[rs, pl.ds(j * SCH, SCH)],
                                accum.at[dbuf.at[s, c * (GCH // SCH) + j]],
                                add=True)

    def pair_body(i, carry):
        do_block(2 * i, 0)
        do_block(2 * i + 1, 1)
        return carry

    lax.fori_loop(0, NB // 2, pair_body, 0)
    plsc.subcore_barrier()

    pltpu.sync_copy(accum.at[pl.ds(sid * ZROWS, ZROWS)],
                    out_hbm.at[cid, pl.ds(sid * ZROWS, ZROWS)])


_sc_gather_scatter = functools.partial(
    pl.kernel,
    out_type=jax.ShapeDtypeStruct((2, NP, M), jnp.float32),
    mesh=plsc.VectorSubcoreMesh(core_axis_name="c", subcore_axis_name="s"),
    scratch_types=[
        pltpu.VMEM((2, EBLK), jnp.int32),
        pltpu.VMEM((2, EBLK // SCH, SCH), jnp.int32),
        pltpu.VMEM((2, GCH, M), jnp.float32),
        pltpu.VMEM_SHARED((NP, M), jnp.float32),
        pltpu.SemaphoreType.DMA,
        pltpu.SemaphoreType.DMA,
        pltpu.SemaphoreType.DMA,
        pltpu.SemaphoreType.DMA,
    ],
    compiler_params=pltpu.CompilerParams(use_tc_tiling_on_sc=False),
)(_sc_gather_scatter_body)


TB = 544


def _mm_body(x_ref, w_ref, o_ref):
    o = jnp.dot(x_ref[...], w_ref[...], preferred_element_type=jnp.float32)
    for t in range(K):
        o_ref[t] = o[:, t * 128:(t + 1) * 128]


def _stats_body(p_ref, o_ref):
    i = pl.program_id(0)

    @pl.when(i == 0)
    def _():
        o_ref[...] = jnp.zeros_like(o_ref)

    rid = lax.broadcasted_iota(jnp.int32, (TB, 128), 0) + i * TB
    msk = (rid < NROW_REAL).astype(jnp.float32)
    s = (p_ref[0] + p_ref[1]) * msk
    o_ref[...] += jnp.concatenate(
        [jnp.sum(s, 0, keepdims=True), jnp.sum(s * s, 0, keepdims=True)], 0)


def _fold16(v):
    acc = v[:, 0:16]
    for j in range(1, 8):
        acc = acc + v[:, 16 * j:16 * j + 16]
    return acc


def _tile128(v):
    return jnp.concatenate([v] * 8, axis=1)


def _apply_body(p_ref, st_ref, g_ref, b_ref, w_ref, o_ref, *, nplanes):
    s = p_ref[0] + p_ref[1]
    mu = _tile128(_fold16(st_ref[0:1, :]) * (1.0 / N))
    ex2 = _tile128(_fold16(st_ref[1:2, :]) * (1.0 / N))
    var = ex2 - mu * mu
    inv = lax.rsqrt(var + 1e-4)
    y = (s - mu) * inv * g_ref[...] + b_ref[...]
    y = jnp.maximum(y, 0.0)
    o = jnp.dot(y, w_ref[...], preferred_element_type=jnp.float32)
    if nplanes == 1:
        o_ref[...] = o
    else:
        for t in range(nplanes):
            o_ref[t] = o[:, t * 128:(t + 1) * 128]


_mm0 = pl.pallas_call(
    _mm_body,
    grid=(NR8 // TB,),
    in_specs=[pl.BlockSpec((TB, 128), lambda i: (i, 0)),
              pl.BlockSpec((128, K * 128), lambda i: (0, 0))],
    out_specs=pl.BlockSpec((K, TB, 128), lambda i: (0, i, 0)),
    out_shape=jax.ShapeDtypeStruct((K, NR8, 128), jnp.float32),
)

_stats = pl.pallas_call(
    _stats_body,
    grid=(NR8 // TB,),
    in_specs=[pl.BlockSpec((2, TB, 128), lambda i: (0, i, 0))],
    out_specs=pl.BlockSpec((2, 128), lambda i: (0, 0)),
    out_shape=jax.ShapeDtypeStruct((2, 128), jnp.float32),
)

_apply_mid = pl.pallas_call(
    functools.partial(_apply_body, nplanes=K),
    grid=(NR8 // TB,),
    in_specs=[pl.BlockSpec((2, TB, 128), lambda i: (0, i, 0)),
              pl.BlockSpec((2, 128), lambda i: (0, 0)),
              pl.BlockSpec((1, 128), lambda i: (0, 0)),
              pl.BlockSpec((1, 128), lambda i: (0, 0)),
              pl.BlockSpec((128, K * 128), lambda i: (0, 0))],
    out_specs=pl.BlockSpec((K, TB, 128), lambda i: (0, i, 0)),
    out_shape=jax.ShapeDtypeStruct((K, NR8, 128), jnp.float32),
)

_apply_fin = pl.pallas_call(
    functools.partial(_apply_body, nplanes=1),
    grid=(NR8 // TB,),
    in_specs=[pl.BlockSpec((2, TB, 128), lambda i: (0, i, 0)),
              pl.BlockSpec((2, 128), lambda i: (0, 0)),
              pl.BlockSpec((1, 128), lambda i: (0, 0)),
              pl.BlockSpec((1, 128), lambda i: (0, 0)),
              pl.BlockSpec((128, 8 * NUM_CLASSES), lambda i: (0, 0))],
    out_specs=pl.BlockSpec((TB, 8 * NUM_CLASSES), lambda i: (i, 0)),
    out_shape=jax.ShapeDtypeStruct((NR8, 8 * NUM_CLASSES), jnp.float32),
)


def _block_diag8(w):
    c = w.shape[1]
    return (jnp.eye(8, dtype=w.dtype)[:, None, :, None]
            * w[None, :, None, :]).reshape(128, 8 * c)


def kernel(feats, edge_index, kernel_id, W0, Ws, gammas, betas, Wlin, blin):
    src = edge_index[0]
    dst = edge_index[1]
    f0 = src * KM + kernel_id * M
    r = f0 // (8 * KM)
    q = f0 % (8 * KM)
    gidx = (q // 128) * NP + r * 8 + (q % 128) // M
    pad = EPAD - E
    gidx_p = jnp.concatenate([gidx, jnp.zeros((pad,), jnp.int32)])
    gidx_p = gidx_p.reshape(NTILES, NB, EBLK)
    dst_p = jnp.concatenate([dst, jnp.full((pad,), DUMMY_ROW, jnp.int32)])
    dst2 = dst_p.reshape(NTILES, NB, EBLK // SCH, SCH)
    zeros_acc = jnp.zeros((NP, M), jnp.float32)

    feats16 = jnp.pad(feats, ((0, NP - N), (0, M - 3))).reshape(NR8, 128)
    w0f = jnp.pad(jnp.transpose(W0, (1, 0, 2)).reshape(3, KM),
                  ((0, M - 3), (0, 0)))
    wbd0 = _block_diag8(w0f)
    wbds = [_block_diag8(jnp.transpose(Ws[i], (1, 0, 2)).reshape(M, KM))
            for i in range(3)]
    wbd_fin = _block_diag8(Wlin)

    z = _mm0(feats16, wbd0)
    out = None
    for i in range(4):
        parts = _sc_gather_scatter(z.reshape(K * NP, M), gidx_p, dst2,
                                   zeros_acc)
        pview = parts.reshape(2, NR8, 128)
        st = _stats(pview)
        g = _tile128(gammas[i].reshape(1, M))
        b = _tile128(betas[i].reshape(1, M))
        if i < 3:
            z = _apply_mid(pview, st, g, b, wbds[i])
        else:
            zf = _apply_fin(pview, st, g, b, wbd_fin)
            out = (zf.reshape(NP, NUM_CLASSES)[:N]
                   + blin.reshape(1, NUM_CLASSES))
    return out


# speedup vs baseline: 39.9211x; 1.0037x over previous
import functools

import jax
import jax.numpy as jnp
from jax import lax
from jax.experimental import pallas as pl
from jax.experimental.pallas import tpu as pltpu
from jax.experimental.pallas import tpu_sc as plsc

N = 100000
E = 1600000
K = 27
M = 16
NUM_CLASSES = 20
KM = K * M

NP = 100096
NR8 = NP // 8
NROW_REAL = N * M // 128

NTILES = 32
SCH = 128
GCH = 512
GPB = 7
EBLK = GPB * GCH
NB = 14
ECHUNK = NB * EBLK
EPAD = NTILES * ECHUNK
DUMMY_ROW = N + 8
ZROWS = NP // 16


def _sc_gather_scatter_body(z_hbm, gidx_hbm, dst_hbm, zeros_hbm, out_hbm,
                            gbuf, dbuf, rows, accum,
                            sr0, sr1, si0, si1, ss0, ss1):
    cid = lax.axis_index("c")
    sid = lax.axis_index("s")
    wid = cid * 16 + sid
    sr = (sr0, sr1)
    si = (si0, si1)
    ss = (ss0, ss1)
    pltpu.sync_copy(zeros_hbm.at[pl.ds(sid * ZROWS, ZROWS)],
                    accum.at[pl.ds(sid * ZROWS, ZROWS)])
    plsc.subcore_barrier()

    NSC = GCH // SCH

    def fire_gather(islot, c, rslot):
        pltpu.async_copy(z_hbm.at[gbuf.at[islot, pl.ds(c * GCH, GCH)]],
                         rows.at[rslot], sr[rslot])

    def wait_gather(islot, c, rslot):
        pltpu.make_async_copy(z_hbm.at[gbuf.at[islot, pl.ds(c * GCH, GCH)]],
                              rows.at[rslot], sr[rslot]).wait()

    def fire_scatters(islot, c, rslot):
        for j in range(NSC):
            pltpu.async_copy(rows.at[rslot, pl.ds(j * SCH, SCH)],
                             accum.at[dbuf.at[islot, c * NSC + j]],
                             ss[rslot], add=True)

    def wait_scatters(islot, c, rslot):
        for j in range(NSC):
            pltpu.make_async_copy(rows.at[rslot, pl.ds(j * SCH, SCH)],
                                  accum.at[dbuf.at[islot, c * NSC + j]],
                                  ss[rslot]).wait()

    pltpu.sync_copy(gidx_hbm.at[wid, 0], gbuf.at[0])
    pltpu.sync_copy(dst_hbm.at[wid, 0], dbuf.at[0])
    fire_gather(0, 0, 0)

    def pair_body(i, carry):
        for m in range(2 * GPB):
            s, c = (0, m) if m < GPB else (1, m - GPB)
            rs = m % 2
            rsn = 1 - rs
            bb = 2 * i + s

            if c == 0:
                @pl.when(bb + 1 < NB)
                def _():
                    pltpu.async_copy(gidx_hbm.at[wid, bb + 1],
                                     gbuf.at[1 - s], si[1 - s])
                    pltpu.async_copy(dst_hbm.at[wid, bb + 1],
                                     dbuf.at[1 - s], si[1 - s])

            if m == 0:
                @pl.when(i > 0)
                def _():
                    wait_scatters(1, GPB - 1, rsn)
                fire_gather(s, c + 1, rsn)
            elif c < GPB - 1:
                prev_s, prev_c = (s, c - 1) if c > 0 else (0, GPB - 1)
                wait_scatters(prev_s, prev_c, rsn)
                fire_gather(s, c + 1, rsn)
            else:
                wait_scatters(s, c - 1, rsn)

                @pl.when(bb + 1 < NB)
                def _():
                    pltpu.make_async_copy(gidx_hbm.at[wid, bb + 1],
                                          gbuf.at[1 - s], si[1 - s]).wait()
                    pltpu.make_async_copy(dst_hbm.at[wid, bb + 1],
                                          dbuf.at[1 - s], si[1 - s]).wait()
                    fire_gather(1 - s, 0, rsn)

            wait_gather(s, c, rs)
            fire_scatters(s, c, rs)
        return carry

    lax.fori_loop(0, NB // 2, pair_body, 0)
    wait_scatters(1, GPB - 1, 1)
    plsc.subcore_barrier()

    pltpu.sync_copy(accum.at[pl.ds(sid * ZROWS, ZROWS)],
                    out_hbm.at[cid, pl.ds(sid * ZROWS, ZROWS)])


_sc_gather_scatter = functools.partial(
    pl.kernel,
    out_type=jax.ShapeDtypeStruct((2, NP, M), jnp.float32),
    mesh=plsc.VectorSubcoreMesh(core_axis_name="c", subcore_axis_name="s"),
    scratch_types=[
        pltpu.VMEM((2, EBLK), jnp.int32),
        pltpu.VMEM((2, EBLK // SCH, SCH), jnp.int32),
        pltpu.VMEM((2, GCH, M), jnp.float32),
        pltpu.VMEM_SHARED((NP, M), jnp.float32),
        pltpu.SemaphoreType.DMA,
        pltpu.SemaphoreType.DMA,
        pltpu.SemaphoreType.DMA,
        pltpu.SemaphoreType.DMA,
        pltpu.SemaphoreType.DMA,
        pltpu.SemaphoreType.DMA,
    ],
    compiler_params=pltpu.CompilerParams(use_tc_tiling_on_sc=False),
)(_sc_gather_scatter_body)


TB = 544


def _mm_body(x_ref, w_ref, o_ref):
    o = jnp.dot(x_ref[...], w_ref[...], preferred_element_type=jnp.float32)
    for t in range(K):
        o_ref[t] = o[:, t * 128:(t + 1) * 128]


def _stats_body(p_ref, o_ref):
    i = pl.program_id(0)

    @pl.when(i == 0)
    def _():
        o_ref[...] = jnp.zeros_like(o_ref)

    rid = lax.broadcasted_iota(jnp.int32, (TB, 128), 0) + i * TB
    msk = (rid < NROW_REAL).astype(jnp.float32)
    s = (p_ref[0] + p_ref[1]) * msk
    o_ref[...] += jnp.concatenate(
        [jnp.sum(s, 0, keepdims=True), jnp.sum(s * s, 0, keepdims=True)], 0)


def _fold16(v):
    acc = v[:, 0:16]
    for j in range(1, 8):
        acc = acc + v[:, 16 * j:16 * j + 16]
    return acc


def _tile128(v):
    return jnp.concatenate([v] * 8, axis=1)


def _apply_body(p_ref, st_ref, g_ref, b_ref, w_ref, o_ref, *, nplanes,
                bias_ref=None):
    s = p_ref[0] + p_ref[1]
    mu = _tile128(_fold16(st_ref[0:1, :]) * (1.0 / N))
    ex2 = _tile128(_fold16(st_ref[1:2, :]) * (1.0 / N))
    var = ex2 - mu * mu
    inv = lax.rsqrt(var + 1e-4)
    y = (s - mu) * inv * g_ref[...] + b_ref[...]
    y = jnp.maximum(y, 0.0)
    o = jnp.dot(y, w_ref[...], preferred_element_type=jnp.float32)
    if nplanes == 1:
        if bias_ref is not None:
            o = o + bias_ref[...]
        o_ref[...] = o
    else:
        for t in range(nplanes):
            o_ref[t] = o[:, t * 128:(t + 1) * 128]


_mm0 = pl.pallas_call(
    _mm_body,
    grid=(NR8 // TB,),
    in_specs=[pl.BlockSpec((TB, 128), lambda i: (i, 0)),
              pl.BlockSpec((128, K * 128), lambda i: (0, 0))],
    out_specs=pl.BlockSpec((K, TB, 128), lambda i: (0, i, 0)),
    out_shape=jax.ShapeDtypeStruct((K, NR8, 128), jnp.float32),
)

_stats = pl.pallas_call(
    _stats_body,
    grid=(NR8 // TB,),
    in_specs=[pl.BlockSpec((2, TB, 128), lambda i: (0, i, 0))],
    out_specs=pl.BlockSpec((2, 128), lambda i: (0, 0)),
    out_shape=jax.ShapeDtypeStruct((2, 128), jnp.float32),
)

_apply_mid = pl.pallas_call(
    functools.partial(_apply_body, nplanes=K),
    grid=(NR8 // TB,),
    in_specs=[pl.BlockSpec((2, TB, 128), lambda i: (0, i, 0)),
              pl.BlockSpec((2, 128), lambda i: (0, 0)),
              pl.BlockSpec((1, 128), lambda i: (0, 0)),
              pl.BlockSpec((1, 128), lambda i: (0, 0)),
              pl.BlockSpec((128, K * 128), lambda i: (0, 0))],
    out_specs=pl.BlockSpec((K, TB, 128), lambda i: (0, i, 0)),
    out_shape=jax.ShapeDtypeStruct((K, NR8, 128), jnp.float32),
)

_apply_fin = pl.pallas_call(
    functools.partial(_apply_body, nplanes=1),
    grid=(NR8 // TB,),
    in_specs=[pl.BlockSpec((2, TB, 128), lambda i: (0, i, 0)),
              pl.BlockSpec((2, 128), lambda i: (0, 0)),
              pl.BlockSpec((1, 128), lambda i: (0, 0)),
              pl.BlockSpec((1, 128), lambda i: (0, 0)),
              pl.BlockSpec((128, 8 * NUM_CLASSES), lambda i: (0, 0))],
    out_specs=pl.BlockSpec((TB, 8 * NUM_CLASSES), lambda i: (i, 0)),
    out_shape=jax.ShapeDtypeStruct((NR8, 8 * NUM_CLASSES), jnp.float32),
)


def _block_diag8(w):
    c = w.shape[1]
    return (jnp.eye(8, dtype=w.dtype)[:, None, :, None]
            * w[None, :, None, :]).reshape(128, 8 * c)


def kernel(feats, edge_index, kernel_id, W0, Ws, gammas, betas, Wlin, blin):
    src = edge_index[0]
    dst = edge_index[1]
    q = (src & 7) * KM + kernel_id * M
    gidx = (q >> 7) * NP + (src >> 3) * 8 + ((q & 127) >> 4)
    pad = EPAD - E
    gidx_p = jnp.concatenate([gidx, jnp.zeros((pad,), jnp.int32)])
    gidx_p = gidx_p.reshape(NTILES, NB, EBLK)
    dst_p = jnp.concatenate([dst, jnp.full((pad,), DUMMY_ROW, jnp.int32)])
    dst2 = dst_p.reshape(NTILES, NB, EBLK // SCH, SCH)
    zeros_acc = jnp.zeros((NP, M), jnp.float32)

    feats16 = jnp.pad(feats, ((0, NP - N), (0, M - 3))).reshape(NR8, 128)
    w0f = jnp.pad(jnp.transpose(W0, (1, 0, 2)).reshape(3, KM),
                  ((0, M - 3), (0, 0)))
    wbd0 = _block_diag8(w0f)
    wbds = [_block_diag8(jnp.transpose(Ws[i], (1, 0, 2)).reshape(M, KM))
            for i in range(3)]
    wbd_fin = _block_diag8(Wlin)

    z = _mm0(feats16, wbd0)
    out = None
    for i in range(4):
        parts = _sc_gather_scatter(z.reshape(K * NP, M), gidx_p, dst2,
                                   zeros_acc)
        pview = parts.reshape(2, NR8, 128)
        st = _stats(pview)
        g = _tile128(gammas[i].reshape(1, M))
        b = _tile128(betas[i].reshape(1, M))
        if i < 3:
            z = _apply_mid(pview, st, g, b, wbds[i])
        else:
            zf = _apply_fin(pview, st, g, b, wbd_fin)
            out = (zf.reshape(NP, NUM_CLASSES)[:N]
                   + blin.reshape(1, NUM_CLASSES))
    return out


# speedup vs baseline: 41.3542x; 1.0359x over previous
import functools

import jax
import jax.numpy as jnp
from jax import lax
from jax.experimental import pallas as pl
from jax.experimental.pallas import tpu as pltpu
from jax.experimental.pallas import tpu_sc as plsc

N = 100000
E = 1600000
K = 27
M = 16
NUM_CLASSES = 20
KM = K * M

NP = 100096
NR8 = NP // 8
NROW_REAL = N * M // 128

NTILES = 32
SCH = 128
GCH = 512
GPB = 7
EBLK = GPB * GCH
NB = 14
ECHUNK = NB * EBLK
EPAD = NTILES * ECHUNK
DUMMY_ROW = N + 8
ZROWS = NP // 16


def _sc_gather_scatter_body(z_hbm, gidx_hbm, dst_hbm, zeros_hbm, out_hbm,
                            gbuf, dbuf, rows, accum,
                            sr0, sr1, si0, si1, ss0, ss1, sz):
    cid = lax.axis_index("c")
    sid = lax.axis_index("s")
    wid = cid * 16 + sid
    sr = (sr0, sr1)
    si = (si0, si1)
    ss = (ss0, ss1)
    pltpu.async_copy(zeros_hbm.at[pl.ds(sid * ZROWS, ZROWS)],
                     accum.at[pl.ds(sid * ZROWS, ZROWS)], sz)

    NSC = GCH // SCH
    RPB = EBLK // SCH

    def fire_gather(islot, c, rslot):
        pltpu.async_copy(
            z_hbm.at[gbuf.at[islot, pl.ds(c * GCH, GCH)]],
            rows.at[rslot], sr[rslot])

    def wait_gather(islot, c, rslot):
        pltpu.make_async_copy(
            z_hbm.at[gbuf.at[islot, pl.ds(c * GCH, GCH)]],
            rows.at[rslot], sr[rslot]).wait()

    def fire_scatters(islot, c, rslot):
        for j in range(NSC):
            pltpu.async_copy(rows.at[rslot, pl.ds(j * SCH, SCH)],
                             accum.at[dbuf.at[islot, c * NSC + j]],
                             ss[rslot], add=True)

    def wait_scatters(islot, c, rslot):
        for j in range(NSC):
            pltpu.make_async_copy(rows.at[rslot, pl.ds(j * SCH, SCH)],
                                  accum.at[dbuf.at[islot, c * NSC + j]],
                                  ss[rslot]).wait()

    base = wid * NB * RPB
    ebase = wid * ECHUNK
    pltpu.sync_copy(gidx_hbm.at[pl.ds(ebase, EBLK)], gbuf.at[0])
    pltpu.sync_copy(dst_hbm.at[pl.ds(base, RPB)], dbuf.at[0])
    fire_gather(0, 0, 0)
    pltpu.make_async_copy(zeros_hbm.at[pl.ds(sid * ZROWS, ZROWS)],
                          accum.at[pl.ds(sid * ZROWS, ZROWS)], sz).wait()
    plsc.subcore_barrier()

    def pair_body(i, carry):
        for m in range(2 * GPB):
            s, c = (0, m) if m < GPB else (1, m - GPB)
            rs = m % 2
            rsn = 1 - rs
            bb = 2 * i + s

            if c == 0:
                @pl.when(bb + 1 < NB)
                def _():
                    pltpu.async_copy(
                        gidx_hbm.at[pl.ds(ebase + (bb + 1) * EBLK, EBLK)],
                        gbuf.at[1 - s], si[1 - s])
                    pltpu.async_copy(
                        dst_hbm.at[pl.ds(base + (bb + 1) * RPB, RPB)],
                        dbuf.at[1 - s], si[1 - s])

            if m == 0:
                @pl.when(i > 0)
                def _():
                    wait_scatters(1, GPB - 1, rsn)
                fire_gather(s, c + 1, rsn)
            elif c < GPB - 1:
                prev_s, prev_c = (s, c - 1) if c > 0 else (0, GPB - 1)
                wait_scatters(prev_s, prev_c, rsn)
                fire_gather(s, c + 1, rsn)
            else:
                wait_scatters(s, c - 1, rsn)

                @pl.when(bb + 1 < NB)
                def _():
                    pltpu.make_async_copy(
                        gidx_hbm.at[pl.ds(ebase + (bb + 1) * EBLK, EBLK)],
                        gbuf.at[1 - s], si[1 - s]).wait()
                    pltpu.make_async_copy(
                        dst_hbm.at[pl.ds(base + (bb + 1) * RPB, RPB)],
                        dbuf.at[1 - s], si[1 - s]).wait()
                    fire_gather(1 - s, 0, rsn)

            wait_gather(s, c, rs)
            fire_scatters(s, c, rs)
        return carry

    lax.fori_loop(0, NB // 2, pair_body, 0)
    wait_scatters(1, GPB - 1, 1)
    plsc.subcore_barrier()

    pltpu.sync_copy(accum.at[pl.ds(sid * ZROWS, ZROWS)],
                    out_hbm.at[cid, pl.ds(sid * ZROWS, ZROWS)])


_sc_gather_scatter = functools.partial(
    pl.kernel,
    out_type=jax.ShapeDtypeStruct((2, NP, M), jnp.float32),
    mesh=plsc.VectorSubcoreMesh(core_axis_name="c", subcore_axis_name="s"),
    scratch_types=[
        pltpu.VMEM((2, EBLK), jnp.int32),
        pltpu.VMEM((2, EBLK // SCH, SCH), jnp.int32),
        pltpu.VMEM((2, GCH, M), jnp.float32),
        pltpu.VMEM_SHARED((NP, M), jnp.float32),
        pltpu.SemaphoreType.DMA,
        pltpu.SemaphoreType.DMA,
        pltpu.SemaphoreType.DMA,
        pltpu.SemaphoreType.DMA,
        pltpu.SemaphoreType.DMA,
        pltpu.SemaphoreType.DMA,
        pltpu.SemaphoreType.DMA,
    ],
    compiler_params=pltpu.CompilerParams(use_tc_tiling_on_sc=False),
)(_sc_gather_scatter_body)


TB = 544


def _mm_body(x_ref, w_ref, o_ref):
    o = jnp.dot(x_ref[...], w_ref[...], preferred_element_type=jnp.float32)
    for t in range(K):
        o_ref[t] = o[:, t * 128:(t + 1) * 128]


def _stats_body(p_ref, o_ref):
    i = pl.program_id(0)

    @pl.when(i == 0)
    def _():
        o_ref[...] = jnp.zeros_like(o_ref)

    rid = lax.broadcasted_iota(jnp.int32, (TB, 128), 0) + i * TB
    msk = (rid < NROW_REAL).astype(jnp.float32)
    s = (p_ref[0] + p_ref[1]) * msk
    o_ref[...] += jnp.concatenate(
        [jnp.sum(s, 0, keepdims=True), jnp.sum(s * s, 0, keepdims=True)], 0)


def _fold16(v):
    acc = v[:, 0:16]
    for j in range(1, 8):
        acc = acc + v[:, 16 * j:16 * j + 16]
    return acc


def _tile128(v):
    return jnp.concatenate([v] * 8, axis=1)


def _bn_relu_128(p_ref, st_ref, g_ref, b_ref):
    s = p_ref[0] + p_ref[1]
    mu = _tile128(_fold16(st_ref[0:1, :]) * (1.0 / N))
    ex2 = _tile128(_fold16(st_ref[1:2, :]) * (1.0 / N))
    var = ex2 - mu * mu
    inv = lax.rsqrt(var + 1e-4)
    y = (s - mu) * inv * g_ref[...] + b_ref[...]
    return jnp.maximum(y, 0.0)


def _apply_body(p_ref, st_ref, g_ref, b_ref, w_ref, o_ref):
    y = _bn_relu_128(p_ref, st_ref, g_ref, b_ref)
    o = jnp.dot(y, w_ref[...], preferred_element_type=jnp.float32)
    for t in range(K):
        o_ref[t] = o[:, t * 128:(t + 1) * 128]


def _apply_fin_body(p_ref, st_ref, g_ref, b_ref, w_ref, bias_ref, o_ref):
    y = _bn_relu_128(p_ref, st_ref, g_ref, b_ref)
    o_ref[...] = (jnp.dot(y, w_ref[...], preferred_element_type=jnp.float32)
                  + bias_ref[...])


_mm0 = pl.pallas_call(
    _mm_body,
    grid=(NR8 // TB,),
    in_specs=[pl.BlockSpec((TB, 128), lambda i: (i, 0)),
              pl.BlockSpec((128, K * 128), lambda i: (0, 0))],
    out_specs=pl.BlockSpec((K, TB, 128), lambda i: (0, i, 0)),
    out_shape=jax.ShapeDtypeStruct((K, NR8, 128), jnp.float32),
)

_stats = pl.pallas_call(
    _stats_body,
    grid=(NR8 // TB,),
    in_specs=[pl.BlockSpec((2, TB, 128), lambda i: (0, i, 0))],
    out_specs=pl.BlockSpec((2, 128), lambda i: (0, 0)),
    out_shape=jax.ShapeDtypeStruct((2, 128), jnp.float32),
)

_apply_mid = pl.pallas_call(
    _apply_body,
    grid=(NR8 // TB,),
    in_specs=[pl.BlockSpec((2, TB, 128), lambda i: (0, i, 0)),
              pl.BlockSpec((2, 128), lambda i: (0, 0)),
              pl.BlockSpec((1, 128), lambda i: (0, 0)),
              pl.BlockSpec((1, 128), lambda i: (0, 0)),
              pl.BlockSpec((128, K * 128), lambda i: (0, 0))],
    out_specs=pl.BlockSpec((K, TB, 128), lambda i: (0, i, 0)),
    out_shape=jax.ShapeDtypeStruct((K, NR8, 128), jnp.float32),
)

_apply_fin = pl.pallas_call(
    _apply_fin_body,
    grid=(NR8 // TB,),
    in_specs=[pl.BlockSpec((2, TB, 128), lambda i: (0, i, 0)),
              pl.BlockSpec((2, 128), lambda i: (0, 0)),
              pl.BlockSpec((1, 128), lambda i: (0, 0)),
              pl.BlockSpec((1, 128), lambda i: (0, 0)),
              pl.BlockSpec((128, 8 * NUM_CLASSES), lambda i: (0, 0)),
              pl.BlockSpec((1, 8 * NUM_CLASSES), lambda i: (0, 0))],
    out_specs=pl.BlockSpec((TB, 8 * NUM_CLASSES), lambda i: (i, 0)),
    out_shape=jax.ShapeDtypeStruct((NR8, 8 * NUM_CLASSES), jnp.float32),
)


def _block_diag8(w):
    c = w.shape[1]
    return (jnp.eye(8, dtype=w.dtype)[:, None, :, None]
            * w[None, :, None, :]).reshape(128, 8 * c)


def kernel(feats, edge_index, kernel_id, W0, Ws, gammas, betas, Wlin, blin):
    src = edge_index[0]
    dst = edge_index[1]
    q = (src & 7) * KM + kernel_id * M
    gidx = (q >> 7) * NP + (src >> 3) * 8 + ((q & 127) >> 4)
    pad = EPAD - E
    gidx_p = jnp.concatenate([gidx, jnp.zeros((pad,), jnp.int32)])
    dst_p = jnp.concatenate([dst, jnp.full((pad,), DUMMY_ROW, jnp.int32)])
    dst2 = dst_p.reshape(EPAD // SCH, SCH)
    zeros_acc = jnp.zeros((NP, M), jnp.float32)

    feats16 = jnp.pad(feats, ((0, NP - N), (0, M - 3))).reshape(NR8, 128)
    w0f = jnp.pad(jnp.transpose(W0, (1, 0, 2)).reshape(3, KM),
                  ((0, M - 3), (0, 0)))
    wbd0 = _block_diag8(w0f)
    wbds = [_block_diag8(jnp.transpose(Ws[i], (1, 0, 2)).reshape(M, KM))
            for i in range(3)]
    wbd_fin = _block_diag8(Wlin)

    z = _mm0(feats16, wbd0)
    out = None
    for i in range(4):
        parts = _sc_gather_scatter(z.reshape(K * NP, M), gidx_p, dst2,
                                   zeros_acc)
        pview = parts.reshape(2, NR8, 128)
        st = _stats(pview)
        g = _tile128(gammas[i].reshape(1, M))
        b = _tile128(betas[i].reshape(1, M))
        if i < 3:
            z = _apply_mid(pview, st, g, b, wbds[i])
        else:
            bias8 = jnp.tile(blin.reshape(1, NUM_CLASSES), (1, 8))
            zf = _apply_fin(pview, st, g, b, wbd_fin, bias8)
            out = zf.reshape(NP, NUM_CLASSES)[:N]
    return out
